# baseline (speedup 1.0000x reference)
import functools

import jax
import jax.numpy as jnp
from jax import lax
from jax.experimental import pallas as pl
from jax.experimental.pallas import tpu as pltpu
from jax.experimental.pallas import tpu_sc as plsc

NC = 2
NS = 16
NW = NC * NS
D = 32
BT = 128


@functools.lru_cache(maxsize=None)
def _gather_call(H: int, B: int):
    assert B == NW * BT
    mesh = plsc.VectorSubcoreMesh(core_axis_name="c", subcore_axis_name="s")

    @functools.partial(
        pl.kernel,
        mesh=mesh,
        out_type=jax.ShapeDtypeStruct((H, D // 8, NW, 8, BT), jnp.float32),
        scratch_types=[
            pltpu.VMEM((H, BT), jnp.int32),
            pltpu.VMEM((BT, D), jnp.float32),
            pltpu.VMEM((BT, D), jnp.float32),
            pltpu.VMEM((D // 8, 8, BT), jnp.float32),
            pltpu.VMEM((D // 8, 8, BT), jnp.float32),
            pltpu.SemaphoreType.DMA,
            pltpu.SemaphoreType.DMA,
        ],
        compiler_params=pltpu.CompilerParams(
            use_tc_tiling_on_sc=False, needs_layout_passes=False
        ),
    )
    def k(idx_hbm, table_hbm, out_hbm, idx_v, rows0, rows1, tr0, tr1, gsem, ssem):
        wid = lax.axis_index("s") * NC + lax.axis_index("c")
        pltpu.sync_copy(idx_hbm.at[:, pl.ds(wid * BT, BT)], idx_v)
        rows = (rows0, rows1)
        trs = (tr0, tr1)

        lane = lax.iota(jnp.int32, 16)

        def gather(h, p):
            return pltpu.make_async_copy(
                table_hbm.at[idx_v.at[h]], rows[p], gsem
            )

        def store(h, p):
            return pltpu.make_async_copy(
                trs[p], out_hbm.at[h, :, wid], ssem
            )

        def transpose(p):
            def body(i, carry):
                vb = lane + i * 16
                for c in range(D):
                    x = plsc.load_gather(
                        rows[p], [vb, jnp.full((16,), c, jnp.int32)]
                    )
                    trs[p][c // 8, c % 8, pl.ds(i * 16, 16)] = x
                return carry

            lax.fori_loop(0, BT // 16, body, 0)

        gather(0, 0).start()
        gather(1, 1).start()
        for p in (0, 1):
            gather(p, p).wait()
            transpose(p)
            store(p, p).start()
            gather(p + 2, p).start()

        def body(i, carry):
            for p in (0, 1):
                h = i * 2 + p
                gather(h, p).wait()
                store(h - 2, p).wait()
                transpose(p)
                store(h, p).start()
                gather(h + 2, p).start()
            return carry

        lax.fori_loop(1, H // 2 - 1, body, 0)

        for p in (0, 1):
            h = H - 2 + p
            gather(h, p).wait()
            store(h - 2, p).wait()
            transpose(p)
            store(h, p).start()
        store(H - 2, 0).wait()
        store(H - 1, 1).wait()

    return k


def kernel(words_pretrained, table):
    b0, hist = words_pretrained.shape
    idx_t = words_pretrained.T.astype(jnp.int32)
    out5 = _gather_call(hist, b0)(idx_t, table)
    return out5.transpose(2, 4, 0, 1, 3).reshape(b0, hist, D)
